# baseline (speedup 1.0000x reference)
import functools

import jax
import jax.numpy as jnp
import numpy as np
from jax import lax
from jax.experimental import pallas as pl
from jax.experimental.pallas import tpu as pltpu
from jax.experimental.pallas import tpu_sc as plsc

B = 4096
S = 200
D = 50
DT = 32
DP = 64
H = 256
NW = 32
RW = B // NW
SA = 104
SB = S - SA

_mesh = plsc.VectorSubcoreMesh(core_axis_name="c", subcore_axis_name="s")


@functools.partial(
    pl.kernel,
    mesh=_mesh,
    compiler_params=pltpu.CompilerParams(
        use_tc_tiling_on_sc=False, needs_layout_passes=False),
    out_type=jax.ShapeDtypeStruct((B, DP), jnp.float32),
    scratch_types=[
        pltpu.VMEM((RW, S), jnp.int32),
        pltpu.VMEM((S, DT), jnp.int32),
        pltpu.VMEM((S, DT), jnp.int32),
        pltpu.VMEM((RW, DP), jnp.float32),
        pltpu.SemaphoreType.DMA,
        pltpu.SemaphoreType.DMA,
    ],
)
def _gather_mean(x_hbm, table_hbm, out_hbm, idx_v, buf0, buf1, out_v, sem0, sem1):
    wid = lax.axis_index("s") * 2 + lax.axis_index("c")
    base = wid * RW
    pltpu.sync_copy(x_hbm.at[pl.ds(base, RW)], idx_v)

    def start(r, buf, sem):
        pltpu.async_copy(table_hbm.at[idx_v.at[r, pl.ds(0, SA)]], buf.at[pl.ds(0, SA)], sem)
        pltpu.async_copy(table_hbm.at[idx_v.at[r, pl.ds(SA, SB)]], buf.at[pl.ds(SA, SB)], sem)

    def drain(buf, sem):
        pltpu.make_async_copy(table_hbm.at[pl.ds(0, S)], buf, sem).wait()

    def accum(buf, r):
        def body(j, accs):
            a0, a1, a2, a3 = accs
            e0, o0 = plsc.unpack(
                plsc.bitcast(buf[j, pl.ds(0, 16)], jnp.bfloat16),
                format=plsc.PackFormat.INTERLEAVED,
                preferred_element_type=jnp.float32)
            e1, o1 = plsc.unpack(
                plsc.bitcast(buf[j, pl.ds(16, 16)], jnp.bfloat16),
                format=plsc.PackFormat.INTERLEAVED,
                preferred_element_type=jnp.float32)
            return (a0 + e0, a1 + e1, a2 + o0, a3 + o1)

        z = jnp.zeros((16,), jnp.float32)
        a = lax.fori_loop(0, S, body, (z, z, z, z), unroll=4)
        scale = jnp.float32(1.0 / S)
        out_v[r, pl.ds(0, 16)] = a[0] * scale
        out_v[r, pl.ds(16, 16)] = a[1] * scale
        out_v[r, pl.ds(32, 16)] = a[2] * scale
        out_v[r, pl.ds(48, 16)] = a[3] * scale

    start(0, buf0, sem0)

    def outer(g, carry):
        r0 = 2 * g
        start(r0 + 1, buf1, sem1)
        drain(buf0, sem0)
        accum(buf0, r0)

        @pl.when(r0 + 2 < RW)
        def _():
            start(r0 + 2, buf0, sem0)

        drain(buf1, sem1)
        accum(buf1, r0 + 1)
        return carry

    lax.fori_loop(0, RW // 2, outer, 0)
    pltpu.sync_copy(out_v, out_hbm.at[pl.ds(base, RW)])


def _mlp_body(h0_ref, w1_ref, b1_ref, w2_ref, b2_ref, out_ref):
    h0 = h0_ref[...]
    h1 = jnp.dot(h0, w1_ref[...], preferred_element_type=jnp.float32) + b1_ref[...]
    h1 = jnp.maximum(h1, 0.0)
    o = jnp.sum(h1 * w2_ref[...], axis=1, keepdims=True) + b2_ref[0, 0]
    out_ref[...] = jax.nn.sigmoid(o)


def kernel(x, table, W1, b1, W2, b2):
    u = jax.lax.bitcast_convert_type(table, jnp.uint32)
    r16 = (u + jnp.uint32(0x7FFF) + ((u >> 16) & jnp.uint32(1))) >> 16
    hi = jnp.pad(r16[:, DT:], ((0, 0), (0, 2 * DT - D)))
    w = r16[:, :DT] | (hi << 16)
    table_i = jax.lax.bitcast_convert_type(w, jnp.int32)
    h0 = _gather_mean(x.astype(jnp.int32), table_i)

    W1p = jnp.pad(W1, ((0, DP - D), (0, 0)))
    out2d = pl.pallas_call(
        _mlp_body,
        out_shape=jax.ShapeDtypeStruct((B, 1), jnp.float32),
    )(h0, W1p, b1.reshape(1, H), W2.reshape(1, H), b2.reshape(1, 1))
    return out2d[:, 0]


# speedup vs baseline: 1.3694x; 1.3694x over previous
import functools

import jax
import jax.numpy as jnp
import numpy as np
from jax import lax
from jax.experimental import pallas as pl
from jax.experimental.pallas import tpu as pltpu
from jax.experimental.pallas import tpu_sc as plsc

B = 4096
S = 200
D = 50
DT = 32
DP = 64
H = 256
NW = 32
RW = B // NW
SA = 104
SB = S - SA

_mesh = plsc.VectorSubcoreMesh(core_axis_name="c", subcore_axis_name="s")


@functools.partial(
    pl.kernel,
    mesh=_mesh,
    compiler_params=pltpu.CompilerParams(
        use_tc_tiling_on_sc=False, needs_layout_passes=False),
    out_type=jax.ShapeDtypeStruct((B, DP), jnp.float32),
    scratch_types=[
        pltpu.VMEM((RW, S), jnp.int32),
        pltpu.VMEM((S, DT), jnp.int32),
        pltpu.VMEM((S, DT), jnp.int32),
        pltpu.VMEM((RW, DP), jnp.float32),
        pltpu.SemaphoreType.DMA,
        pltpu.SemaphoreType.DMA,
    ],
)
def _gather_mean(x_hbm, table_hbm, out_hbm, idx_v, buf0, buf1, out_v, sem0, sem1):
    wid = lax.axis_index("s") * 2 + lax.axis_index("c")
    base = wid * RW
    pltpu.sync_copy(x_hbm.at[pl.ds(base, RW)], idx_v)

    def start(r, buf, sem):
        pltpu.async_copy(table_hbm.at[idx_v.at[r, pl.ds(0, SA)]], buf.at[pl.ds(0, SA)], sem)
        pltpu.async_copy(table_hbm.at[idx_v.at[r, pl.ds(SA, SB)]], buf.at[pl.ds(SA, SB)], sem)

    def drain(buf, sem):
        pltpu.make_async_copy(table_hbm.at[pl.ds(0, S)], buf, sem).wait()

    def accum(buf, r):
        def body(j, accs):
            a0, a1, a2, a3 = accs
            e0, o0 = plsc.unpack(
                plsc.bitcast(buf[j, pl.ds(0, 16)], jnp.bfloat16),
                format=plsc.PackFormat.INTERLEAVED,
                preferred_element_type=jnp.float32)
            e1, o1 = plsc.unpack(
                plsc.bitcast(buf[j, pl.ds(16, 16)], jnp.bfloat16),
                format=plsc.PackFormat.INTERLEAVED,
                preferred_element_type=jnp.float32)
            return (a0 + e0, a1 + e1, a2 + o0, a3 + o1)

        z = jnp.zeros((16,), jnp.float32)
        a = lax.fori_loop(0, S, body, (z, z, z, z), unroll=4)
        scale = jnp.float32(1.0 / S)
        out_v[r, pl.ds(0, 16)] = a[0] * scale
        out_v[r, pl.ds(16, 16)] = a[1] * scale
        out_v[r, pl.ds(32, 16)] = a[2] * scale
        out_v[r, pl.ds(48, 16)] = a[3] * scale

    start(0, buf0, sem0)

    def outer(g, carry):
        r0 = 2 * g
        start(r0 + 1, buf1, sem1)
        drain(buf0, sem0)
        accum(buf0, r0)

        @pl.when(r0 + 2 < RW)
        def _():
            start(r0 + 2, buf0, sem0)

        drain(buf1, sem1)
        accum(buf1, r0 + 1)
        return carry

    lax.fori_loop(0, RW // 2, outer, 0)
    pltpu.sync_copy(out_v, out_hbm.at[pl.ds(base, RW)])


RB = 8192


def _pack_body(t_ref, o_ref):
    u = jax.lax.bitcast_convert_type(t_ref[...], jnp.uint32)
    r16 = (u + jnp.uint32(0x7FFF) + ((u >> 16) & jnp.uint32(1))) >> 16
    hi = jnp.concatenate(
        [r16[:, DT:], jnp.zeros((t_ref.shape[0], 2 * DT - D), jnp.uint32)],
        axis=1)
    o_ref[...] = jax.lax.bitcast_convert_type(r16[:, :DT] | (hi << 16), jnp.int32)


def _mlp_body(h0_ref, w1_ref, b1_ref, w2_ref, b2_ref, out_ref):
    h0 = h0_ref[...]
    h1 = jnp.dot(h0, w1_ref[...], preferred_element_type=jnp.float32) + b1_ref[...]
    h1 = jnp.maximum(h1, 0.0)
    o = jnp.sum(h1 * w2_ref[...], axis=1, keepdims=True) + b2_ref[0, 0]
    out_ref[...] = jax.nn.sigmoid(o)


def kernel(x, table, W1, b1, W2, b2):
    nv = table.shape[0]
    grid = (nv + RB - 1) // RB
    table_i = pl.pallas_call(
        _pack_body,
        grid=(grid,),
        in_specs=[pl.BlockSpec((RB, D), lambda i: (i, 0))],
        out_specs=pl.BlockSpec((RB, DT), lambda i: (i, 0)),
        out_shape=jax.ShapeDtypeStruct((nv, DT), jnp.int32),
    )(table)
    h0 = _gather_mean(x.astype(jnp.int32), table_i)

    W1p = jnp.pad(W1, ((0, DP - D), (0, 0)))
    out2d = pl.pallas_call(
        _mlp_body,
        out_shape=jax.ShapeDtypeStruct((B, 1), jnp.float32),
    )(h0, W1p, b1.reshape(1, H), W2.reshape(1, H), b2.reshape(1, 1))
    return out2d[:, 0]


# speedup vs baseline: 1.3814x; 1.0088x over previous
import functools

import jax
import jax.numpy as jnp
import numpy as np
from jax import lax
from jax.experimental import pallas as pl
from jax.experimental.pallas import tpu as pltpu
from jax.experimental.pallas import tpu_sc as plsc

B = 4096
S = 200
D = 50
DT = 32
DP = 64
H = 256
NW = 32
RW = B // NW
SA = 104
SB = S - SA
SX = 256

_mesh = plsc.VectorSubcoreMesh(core_axis_name="c", subcore_axis_name="s")


@functools.partial(
    pl.kernel,
    mesh=_mesh,
    compiler_params=pltpu.CompilerParams(
        use_tc_tiling_on_sc=False, needs_layout_passes=False),
    out_type=jax.ShapeDtypeStruct((B, DP), jnp.float32),
    scratch_types=[
        pltpu.VMEM((RW, SX), jnp.int32),
        pltpu.VMEM((S, DT), jnp.int32),
        pltpu.VMEM((S, DT), jnp.int32),
        pltpu.VMEM((RW, DP), jnp.float32),
        pltpu.SemaphoreType.DMA,
        pltpu.SemaphoreType.DMA,
    ],
)
def _gather_mean(x_hbm, table_hbm, out_hbm, idx_v, buf0, buf1, out_v, sem0, sem1):
    wid = lax.axis_index("s") * 2 + lax.axis_index("c")
    base = wid * RW
    pltpu.sync_copy(x_hbm.at[pl.ds(base, RW)], idx_v)

    def start(r, buf, sem):
        pltpu.async_copy(table_hbm.at[idx_v.at[r, pl.ds(0, SA)]], buf.at[pl.ds(0, SA)], sem)
        pltpu.async_copy(table_hbm.at[idx_v.at[r, pl.ds(SA, SB)]], buf.at[pl.ds(SA, SB)], sem)

    def drain(buf, sem):
        pltpu.make_async_copy(table_hbm.at[pl.ds(0, S)], buf, sem).wait()

    def accum(buf, r):
        def body(j, accs):
            a0, a1, a2, a3 = accs
            e0, o0 = plsc.unpack(
                plsc.bitcast(buf[j, pl.ds(0, 16)], jnp.bfloat16),
                format=plsc.PackFormat.INTERLEAVED,
                preferred_element_type=jnp.float32)
            e1, o1 = plsc.unpack(
                plsc.bitcast(buf[j, pl.ds(16, 16)], jnp.bfloat16),
                format=plsc.PackFormat.INTERLEAVED,
                preferred_element_type=jnp.float32)
            return (a0 + e0, a1 + e1, a2 + o0, a3 + o1)

        z = jnp.zeros((16,), jnp.float32)
        a = lax.fori_loop(0, S, body, (z, z, z, z), unroll=4)
        scale = jnp.float32(1.0 / S)
        out_v[r, pl.ds(0, 16)] = a[0] * scale
        out_v[r, pl.ds(16, 16)] = a[1] * scale
        out_v[r, pl.ds(32, 16)] = a[2] * scale
        out_v[r, pl.ds(48, 16)] = a[3] * scale

    start(0, buf0, sem0)

    def outer(g, carry):
        r0 = 2 * g
        start(r0 + 1, buf1, sem1)
        drain(buf0, sem0)
        accum(buf0, r0)

        @pl.when(r0 + 2 < RW)
        def _():
            start(r0 + 2, buf0, sem0)

        drain(buf1, sem1)
        accum(buf1, r0 + 1)
        return carry

    lax.fori_loop(0, RW // 2, outer, 0)
    pltpu.sync_copy(out_v, out_hbm.at[pl.ds(base, RW)])


RB = 8192


def _pack_body(t_ref, o_ref):
    u = jax.lax.bitcast_convert_type(t_ref[...], jnp.uint32)
    r16 = (u + jnp.uint32(0x7FFF) + ((u >> 16) & jnp.uint32(1))) >> 16
    hi = jnp.concatenate(
        [r16[:, DT:], jnp.zeros((t_ref.shape[0], 2 * DT - D), jnp.uint32)],
        axis=1)
    o_ref[...] = jax.lax.bitcast_convert_type(r16[:, :DT] | (hi << 16), jnp.int32)


def _mlp_body(h0_ref, w1_ref, b1_ref, w2_ref, b2_ref, out_ref):
    h0 = h0_ref[...]
    h1 = jnp.dot(h0, w1_ref[...], preferred_element_type=jnp.float32) + b1_ref[...]
    h1 = jnp.maximum(h1, 0.0)
    o = jnp.sum(h1 * w2_ref[...], axis=1, keepdims=True) + b2_ref[0, 0]
    out_ref[...] = jax.nn.sigmoid(o)


def kernel(x, table, W1, b1, W2, b2):
    nv = table.shape[0]
    grid = (nv + RB - 1) // RB
    table_i = pl.pallas_call(
        _pack_body,
        grid=(grid,),
        in_specs=[pl.BlockSpec((RB, D), lambda i: (i, 0))],
        out_specs=pl.BlockSpec((RB, DT), lambda i: (i, 0)),
        out_shape=jax.ShapeDtypeStruct((nv, DT), jnp.int32),
    )(table)
    x_p = jnp.pad(x.astype(jnp.int32), ((0, 0), (0, SX - S)))
    h0 = _gather_mean(x_p, table_i)

    W1p = jnp.pad(W1, ((0, DP - D), (0, 0)))
    out2d = pl.pallas_call(
        _mlp_body,
        out_shape=jax.ShapeDtypeStruct((B, 1), jnp.float32),
    )(h0, W1p, b1.reshape(1, H), W2.reshape(1, H), b2.reshape(1, 1))
    return out2d[:, 0]


# speedup vs baseline: 1.3864x; 1.0037x over previous
import functools

import jax
import jax.numpy as jnp
import numpy as np
from jax import lax
from jax.experimental import pallas as pl
from jax.experimental.pallas import tpu as pltpu
from jax.experimental.pallas import tpu_sc as plsc

B = 4096
S = 200
D = 50
DT = 32
DP = 64
H = 256
NW = 32
RW = B // NW
SA = 104
SB = S - SA
SX = 256

_mesh = plsc.VectorSubcoreMesh(core_axis_name="c", subcore_axis_name="s")


@functools.partial(
    pl.kernel,
    mesh=_mesh,
    compiler_params=pltpu.CompilerParams(
        use_tc_tiling_on_sc=False, needs_layout_passes=False),
    out_type=jax.ShapeDtypeStruct((B, DP), jnp.float32),
    scratch_types=[
        pltpu.VMEM((RW, SX), jnp.int32),
        pltpu.VMEM((S, DT), jnp.int32),
        pltpu.VMEM((S, DT), jnp.int32),
        pltpu.VMEM((RW, DP), jnp.float32),
        pltpu.SemaphoreType.DMA,
        pltpu.SemaphoreType.DMA,
    ],
)
def _gather_mean(x_hbm, table_hbm, out_hbm, idx_v, buf0, buf1, out_v, sem0, sem1):
    wid = lax.axis_index("s") * 2 + lax.axis_index("c")
    base = wid * RW
    pltpu.sync_copy(x_hbm.at[pl.ds(base, RW)], idx_v)

    def start(r, buf, sem):
        pltpu.async_copy(table_hbm.at[idx_v.at[r, pl.ds(0, SA)]], buf.at[pl.ds(0, SA)], sem)
        pltpu.async_copy(table_hbm.at[idx_v.at[r, pl.ds(SA, SB)]], buf.at[pl.ds(SA, SB)], sem)

    def drain(buf, sem):
        pltpu.make_async_copy(table_hbm.at[pl.ds(0, S)], buf, sem).wait()

    def accum(buf, r):
        def body(j, accs):
            a0, a1, a2, a3 = accs
            w0 = buf[j, pl.ds(0, 16)]
            w1 = buf[j, pl.ds(16, 16)]
            lo0 = plsc.bitcast(w0 << 16, jnp.float32)
            lo1 = plsc.bitcast(w1 << 16, jnp.float32)
            hi0 = plsc.bitcast(w0 & jnp.int32(-65536), jnp.float32)
            hi1 = plsc.bitcast(w1 & jnp.int32(-65536), jnp.float32)
            return (a0 + lo0, a1 + lo1, a2 + hi0, a3 + hi1)

        z = jnp.zeros((16,), jnp.float32)
        a = lax.fori_loop(0, S, body, (z, z, z, z), unroll=4)
        scale = jnp.float32(1.0 / S)
        out_v[r, pl.ds(0, 16)] = a[0] * scale
        out_v[r, pl.ds(16, 16)] = a[1] * scale
        out_v[r, pl.ds(32, 16)] = a[2] * scale
        out_v[r, pl.ds(48, 16)] = a[3] * scale

    start(0, buf0, sem0)

    def outer(g, carry):
        r0 = 2 * g
        start(r0 + 1, buf1, sem1)
        drain(buf0, sem0)
        accum(buf0, r0)

        @pl.when(r0 + 2 < RW)
        def _():
            start(r0 + 2, buf0, sem0)

        drain(buf1, sem1)
        accum(buf1, r0 + 1)
        return carry

    lax.fori_loop(0, RW // 2, outer, 0)
    pltpu.sync_copy(out_v, out_hbm.at[pl.ds(base, RW)])


RB = 8192


def _pack_body(t_ref, o_ref):
    u = jax.lax.bitcast_convert_type(t_ref[...], jnp.uint32)
    r16 = (u + jnp.uint32(0x8000)) >> 16
    hi = jnp.concatenate(
        [r16[:, DT:], jnp.zeros((t_ref.shape[0], 2 * DT - D), jnp.uint32)],
        axis=1)
    o_ref[...] = jax.lax.bitcast_convert_type(r16[:, :DT] | (hi << 16), jnp.int32)


def _mlp_body(h0_ref, w1_ref, b1_ref, w2_ref, b2_ref, out_ref):
    h0 = h0_ref[...]
    h1 = jnp.dot(h0, w1_ref[...], preferred_element_type=jnp.float32) + b1_ref[...]
    h1 = jnp.maximum(h1, 0.0)
    o = jnp.sum(h1 * w2_ref[...], axis=1, keepdims=True) + b2_ref[0, 0]
    out_ref[...] = jax.nn.sigmoid(o)


def kernel(x, table, W1, b1, W2, b2):
    nv = table.shape[0]
    grid = (nv + RB - 1) // RB
    table_i = pl.pallas_call(
        _pack_body,
        grid=(grid,),
        in_specs=[pl.BlockSpec((RB, D), lambda i: (i, 0))],
        out_specs=pl.BlockSpec((RB, DT), lambda i: (i, 0)),
        out_shape=jax.ShapeDtypeStruct((nv, DT), jnp.int32),
    )(table)
    x_p = jnp.pad(x.astype(jnp.int32), ((0, 0), (0, SX - S)))
    h0 = _gather_mean(x_p, table_i)

    W1p = jnp.pad(W1, ((0, DP - D), (0, 0)))
    out2d = pl.pallas_call(
        _mlp_body,
        out_shape=jax.ShapeDtypeStruct((B, 1), jnp.float32),
    )(h0, W1p, b1.reshape(1, H), W2.reshape(1, H), b2.reshape(1, 1))
    return out2d[:, 0]


# speedup vs baseline: 1.3873x; 1.0006x over previous
import functools

import jax
import jax.numpy as jnp
import numpy as np
from jax import lax
from jax.experimental import pallas as pl
from jax.experimental.pallas import tpu as pltpu
from jax.experimental.pallas import tpu_sc as plsc

B = 4096
S = 200
D = 50
DT = 32
DP = 64
H = 256
NW = 32
RW = B // NW
SA = 104
SB = S - SA
SX = 256

_mesh = plsc.VectorSubcoreMesh(core_axis_name="c", subcore_axis_name="s")


@functools.partial(
    pl.kernel,
    mesh=_mesh,
    compiler_params=pltpu.CompilerParams(
        use_tc_tiling_on_sc=False, needs_layout_passes=False),
    out_type=jax.ShapeDtypeStruct((B, DP), jnp.float32),
    scratch_types=[
        pltpu.VMEM((RW, SX), jnp.int32),
        pltpu.VMEM((S, DT), jnp.int32),
        pltpu.VMEM((S, DT), jnp.int32),
        pltpu.VMEM((RW, DP), jnp.float32),
        pltpu.SemaphoreType.DMA,
        pltpu.SemaphoreType.DMA,
    ],
)
def _gather_mean(x_hbm, table_hbm, out_hbm, idx_v, buf0, buf1, out_v, sem0, sem1):
    wid = lax.axis_index("s") * 2 + lax.axis_index("c")
    base = wid * RW
    pltpu.sync_copy(x_hbm.at[pl.ds(base, RW)], idx_v)

    def start(r, buf, sem):
        pltpu.async_copy(table_hbm.at[idx_v.at[r, pl.ds(0, SA)]], buf.at[pl.ds(0, SA)], sem)
        pltpu.async_copy(table_hbm.at[idx_v.at[r, pl.ds(SA, SB)]], buf.at[pl.ds(SA, SB)], sem)

    def drain(buf, sem):
        pltpu.make_async_copy(table_hbm.at[pl.ds(0, S)], buf, sem).wait()

    def accum(buf, r):
        def body(j, accs):
            a0, a1, a2, a3 = accs
            w0 = buf[j, pl.ds(0, 16)]
            w1 = buf[j, pl.ds(16, 16)]
            lo0 = plsc.bitcast(w0 << 16, jnp.float32)
            lo1 = plsc.bitcast(w1 << 16, jnp.float32)
            hi0 = plsc.bitcast(w0 & jnp.int32(-65536), jnp.float32)
            hi1 = plsc.bitcast(w1 & jnp.int32(-65536), jnp.float32)
            return (a0 + lo0, a1 + lo1, a2 + hi0, a3 + hi1)

        z = jnp.zeros((16,), jnp.float32)
        a = lax.fori_loop(0, S, body, (z, z, z, z), unroll=8)
        scale = jnp.float32(1.0 / S)
        out_v[r, pl.ds(0, 16)] = a[0] * scale
        out_v[r, pl.ds(16, 16)] = a[1] * scale
        out_v[r, pl.ds(32, 16)] = a[2] * scale
        out_v[r, pl.ds(48, 16)] = a[3] * scale

    start(0, buf0, sem0)

    def outer(g, carry):
        r0 = 2 * g
        start(r0 + 1, buf1, sem1)
        drain(buf0, sem0)
        accum(buf0, r0)

        @pl.when(r0 + 2 < RW)
        def _():
            start(r0 + 2, buf0, sem0)

        drain(buf1, sem1)
        accum(buf1, r0 + 1)
        return carry

    lax.fori_loop(0, RW // 2, outer, 0)
    pltpu.sync_copy(out_v, out_hbm.at[pl.ds(base, RW)])


RB = 16384


def _pack_body(t_ref, o_ref):
    u = jax.lax.bitcast_convert_type(t_ref[...], jnp.uint32)
    r16 = (u + jnp.uint32(0x8000)) >> 16
    hi = jnp.concatenate(
        [r16[:, DT:], jnp.zeros((t_ref.shape[0], 2 * DT - D), jnp.uint32)],
        axis=1)
    o_ref[...] = jax.lax.bitcast_convert_type(r16[:, :DT] | (hi << 16), jnp.int32)


def _mlp_body(h0_ref, w1_ref, b1_ref, w2_ref, b2_ref, out_ref):
    h0 = h0_ref[...]
    h1 = jnp.dot(h0, w1_ref[...], preferred_element_type=jnp.float32) + b1_ref[...]
    h1 = jnp.maximum(h1, 0.0)
    o = jnp.sum(h1 * w2_ref[...], axis=1, keepdims=True) + b2_ref[0, 0]
    out_ref[...] = jax.nn.sigmoid(o)


def kernel(x, table, W1, b1, W2, b2):
    nv = table.shape[0]
    grid = (nv + RB - 1) // RB
    table_i = pl.pallas_call(
        _pack_body,
        grid=(grid,),
        in_specs=[pl.BlockSpec((RB, D), lambda i: (i, 0))],
        out_specs=pl.BlockSpec((RB, DT), lambda i: (i, 0)),
        out_shape=jax.ShapeDtypeStruct((nv, DT), jnp.int32),
    )(table)
    x_p = jnp.pad(x.astype(jnp.int32), ((0, 0), (0, SX - S)))
    h0 = _gather_mean(x_p, table_i)

    W1p = jnp.pad(W1, ((0, DP - D), (0, 0)))
    out2d = pl.pallas_call(
        _mlp_body,
        out_shape=jax.ShapeDtypeStruct((B, 1), jnp.float32),
    )(h0, W1p, b1.reshape(1, H), W2.reshape(1, H), b2.reshape(1, 1))
    return out2d[:, 0]
